# Initial kernel scaffold; baseline (speedup 1.0000x reference)
import jax
import jax.numpy as jnp
from jax.experimental import pallas as pl


def kernel(x, p, w, bias):
    raise NotImplementedError("write your pallas kernel here")


# speedup vs baseline: 11.0627x; 11.0627x over previous
import functools

import jax
import jax.numpy as jnp
from jax import lax
from jax.experimental import pallas as pl
from jax.experimental.pallas import tpu as pltpu

_VMEM_LIMIT = 48 * 1024 * 1024


def _fps_kernel(pc_ref, sel_ref, mind_ref, *, n, n_new):
    px = pc_ref[0]
    py = pc_ref[1]
    pz = pc_ref[2]
    bb = px.shape[0]
    col = lax.broadcasted_iota(jnp.int32, (bb, n), 1)
    ncol = lax.broadcasted_iota(jnp.int32, (bb, n_new), 1)
    mind_ref[...] = jnp.full((bb, n), jnp.inf, jnp.float32)
    sel_ref[...] = jnp.zeros((bb, n_new), jnp.int32)

    def body(i, carry):
        lx, ly, lz = carry
        dx = px - lx
        dy = py - ly
        dz = pz - lz
        d = (dx * dx + dy * dy) + dz * dz
        nmin = jnp.minimum(mind_ref[...], d)
        mind_ref[...] = nmin
        nxt = jnp.argmax(nmin, axis=-1, keepdims=True)
        sel_ref[...] = jnp.where(ncol == i, nxt.astype(jnp.int32), sel_ref[...])
        sm = col == nxt
        nlx = jnp.sum(jnp.where(sm, px, 0.0), axis=-1, keepdims=True)
        nly = jnp.sum(jnp.where(sm, py, 0.0), axis=-1, keepdims=True)
        nlz = jnp.sum(jnp.where(sm, pz, 0.0), axis=-1, keepdims=True)
        return (nlx, nly, nlz)

    lax.fori_loop(1, n_new, body, (px[:, 0:1], py[:, 0:1], pz[:, 0:1]))


def _fps(p, n_new):
    B, n, _ = p.shape
    ncores = 2 if B % 2 == 0 and (B // 2) % 8 == 0 else 1
    bb = B // ncores
    pc = jnp.moveaxis(p, -1, 0)
    sel = pl.pallas_call(
        functools.partial(_fps_kernel, n=n, n_new=n_new),
        out_shape=jax.ShapeDtypeStruct((B, n_new), jnp.int32),
        grid_spec=pltpu.PrefetchScalarGridSpec(
            num_scalar_prefetch=0,
            grid=(ncores,),
            in_specs=[pl.BlockSpec((3, bb, n), lambda i: (0, i, 0))],
            out_specs=pl.BlockSpec((bb, n_new), lambda i: (i, 0)),
            scratch_shapes=[pltpu.VMEM((bb, n), jnp.float32)],
        ),
        compiler_params=pltpu.CompilerParams(
            dimension_semantics=("parallel",), vmem_limit_bytes=_VMEM_LIMIT),
    )(pc)
    return jnp.sort(sel, axis=-1)


def _knn_kernel(ps_ref, pt_ref, idx_ref, *, n, k):
    ps = ps_ref[0]
    pt = pt_ref[0]
    tq = ps.shape[0]
    cross = jnp.dot(ps, pt, preferred_element_type=jnp.float32)
    ps2 = jnp.sum(ps * ps, axis=-1, keepdims=True)
    p2 = jnp.sum(pt * pt, axis=0, keepdims=True)
    d2 = ps2 + p2 - 2.0 * cross

    colf = lax.broadcasted_iota(jnp.int32, (tq, n), 1).astype(jnp.float32)
    lanek = lax.broadcasted_iota(jnp.int32, (tq, k), 1)
    big = jnp.float32(3e9)
    acc = jnp.zeros((tq, k), jnp.float32)
    for t in range(k):
        m = jnp.min(d2, axis=-1, keepdims=True)
        sel = jnp.min(jnp.where(d2 == m, colf, big), axis=-1, keepdims=True)
        acc = jnp.where(lanek == t, sel, acc)
        d2 = jnp.where(colf == sel, jnp.inf, d2)
    idx_ref[0] = acc.astype(jnp.int32)


def _knn(p_sub, p, k):
    B, n_new, _ = p_sub.shape
    _, n, _ = p.shape
    tq = min(256, n_new)
    ps = jnp.pad(p_sub, ((0, 0), (0, 0), (0, 5)))
    pt = jnp.pad(jnp.swapaxes(p, 1, 2), ((0, 0), (0, 5), (0, 0)))
    return pl.pallas_call(
        functools.partial(_knn_kernel, n=n, k=k),
        out_shape=jax.ShapeDtypeStruct((B, n_new, k), jnp.int32),
        grid_spec=pltpu.PrefetchScalarGridSpec(
            num_scalar_prefetch=0,
            grid=(B, n_new // tq),
            in_specs=[
                pl.BlockSpec((1, tq, 8), lambda b, i: (b, i, 0)),
                pl.BlockSpec((1, 8, n), lambda b, i: (b, 0, 0)),
            ],
            out_specs=pl.BlockSpec((1, tq, k), lambda b, i: (b, i, 0)),
        ),
        compiler_params=pltpu.CompilerParams(
            dimension_semantics=("parallel", "parallel"),
            vmem_limit_bytes=_VMEM_LIMIT),
    )(ps, pt)


def _mgp_kernel(x_ref, w_ref, b_ref, idx_ref, z_ref, y_s, *, tq, k):
    @pl.when(pl.program_id(1) == 0)
    def _mlp():
        y = jnp.dot(x_ref[0], w_ref[...], preferred_element_type=jnp.float32)
        y_s[...] = jnp.maximum(y + b_ref[...], 0.0)

    d = y_s.shape[-1]
    iota8 = lax.broadcasted_iota(jnp.int32, (8, d), 0)
    neg = jnp.float32(-jnp.inf)

    def group(g, _):
        out = jnp.full((8, d), neg, jnp.float32)
        for qq in range(8):
            q = g * 8 + qq
            best = jnp.full((8, d), neg, jnp.float32)
            for j in range(k):
                r = idx_ref[0, q, j]
                chunk = y_s[pl.ds(pl.multiple_of((r >> 3) << 3, 8), 8), :]
                best = jnp.maximum(best, jnp.where(iota8 == (r & 7), chunk, neg))
            best = jnp.maximum(best, pltpu.roll(best, 4, axis=0))
            best = jnp.maximum(best, pltpu.roll(best, 2, axis=0))
            best = jnp.maximum(best, pltpu.roll(best, 1, axis=0))
            out = jnp.where(iota8 == qq, best, out)
        z_ref[0, pl.ds(g * 8, 8), :] = out
        return 0

    lax.fori_loop(0, tq // 8, group, 0)


def _mlp_gather_pool(x, w, bias, idx, k):
    B, n, d_in = x.shape
    d_out = w.shape[1]
    n_new = idx.shape[1]
    tq = min(256, n_new)
    b2 = bias.reshape(1, d_out)
    return pl.pallas_call(
        functools.partial(_mgp_kernel, tq=tq, k=k),
        out_shape=jax.ShapeDtypeStruct((B, n_new, d_out), jnp.float32),
        grid_spec=pltpu.PrefetchScalarGridSpec(
            num_scalar_prefetch=0,
            grid=(B, n_new // tq),
            in_specs=[
                pl.BlockSpec((1, n, d_in), lambda b, i: (b, 0, 0)),
                pl.BlockSpec((d_in, d_out), lambda b, i: (0, 0)),
                pl.BlockSpec((1, d_out), lambda b, i: (0, 0)),
                pl.BlockSpec((1, tq, k), lambda b, i: (b, i, 0),
                             memory_space=pltpu.SMEM),
            ],
            out_specs=pl.BlockSpec((1, tq, d_out), lambda b, i: (b, i, 0)),
            scratch_shapes=[pltpu.VMEM((n, d_out), jnp.float32)],
        ),
        compiler_params=pltpu.CompilerParams(
            dimension_semantics=("parallel", "arbitrary"),
            vmem_limit_bytes=_VMEM_LIMIT),
    )(x, w, b2, idx)


def kernel(x, p, w, bias, *, factor=2, knn_k=8):
    B, n, d_in = x.shape
    n_new = -(-n // factor)

    sub_idx = _fps(p, n_new)
    p_sub = jnp.take_along_axis(p, sub_idx[..., None], axis=1)
    knn_idx = _knn(p_sub, p, knn_k)
    z = _mlp_gather_pool(x, w, bias, knn_idx, knn_k)
    return z, p_sub, knn_idx
